# baseline (speedup 1.0000x reference)
import functools

import jax
import jax.numpy as jnp
from jax import lax
from jax.experimental import pallas as pl
from jax.experimental.pallas import tpu as pltpu
from jax.experimental.pallas import tpu_sc as plsc

B, NT, NS = 16, 4096, 1024
CT, CS = 256, 512
C1, C2 = 256, 256
TB = 1024
TBC = 2048

NW = 32
ROWS = B * NT
RPW = ROWS // NW
CH = 32
NCHUNK = RPW // CH


def _g_kernel(fs_ref, w1a_ref, g_ref):
    g_ref[0] = jnp.dot(fs_ref[0], w1a_ref[...],
                       preferred_element_type=jnp.float32)


def _nn_kernel(xt_ref, xst_ref, idx_ref, w_ref):
    xt = xt_ref[0]
    xst = xst_ref[0]
    diff0 = xt[:, 0:1] - xst[0:1, :]
    d2 = diff0 * diff0
    diff1 = xt[:, 1:2] - xst[1:2, :]
    d2 = d2 + diff1 * diff1
    diff2 = xt[:, 2:3] - xst[2:3, :]
    d2 = d2 + diff2 * diff2

    d2w = d2
    ohs = []
    recips = []
    for k in range(3):
        m = jnp.min(d2w, axis=1, keepdims=True)
        oh = d2w == m
        if k < 2:
            d2w = jnp.where(oh, jnp.float32(jnp.inf), d2w)
        recips.append(jax.lax.rsqrt(jnp.maximum(m, 1e-20)))
        ohs.append(oh)
    r0, r1, r2 = recips
    norm = r0 + r1 + r2
    rn = 1.0 / norm
    ws = (r0 + r1 + r2) * rn + 1e-6
    c = rn / ws

    b = pl.program_id(0)
    lane = lax.broadcasted_iota(jnp.int32, (TB, NS), 1)
    for k, (oh, r) in enumerate(zip(ohs, recips)):
        idx_i = jnp.max(jnp.where(oh, lane, 0), axis=1, keepdims=True)
        idx_ref[0, :, k:k + 1] = idx_i + b * NS
        w_ref[0, :, k * 16:(k + 1) * 16] = jnp.broadcast_to(r * c, (TB, 16))


def _gather_kernel(g_hbm, idx_hbm, wts_hbm, out_hbm,
                   idx_v, w_v, rows_v, out_v, sem):
    wid = lax.axis_index("s") * 2 + lax.axis_index("c")

    def chunk_body(cidx, carry):
        base = wid * RPW + cidx * CH
        pltpu.sync_copy(idx_hbm.at[pl.ds(base * 3, CH * 3)], idx_v)
        pltpu.sync_copy(wts_hbm.at[pl.ds(base * 3, CH * 3)], w_v)
        pltpu.async_copy(g_hbm.at[idx_v], rows_v, sem).wait()

        def row_body(i, c2):
            w0 = w_v[3 * i, :]
            w1 = w_v[3 * i + 1, :]
            w2 = w_v[3 * i + 2, :]
            for j in range(C1 // 16):
                sl = pl.ds(j * 16, 16)
                a = rows_v[3 * i, sl] * w0
                a = a + rows_v[3 * i + 1, sl] * w1
                a = a + rows_v[3 * i + 2, sl] * w2
                out_v[i, sl] = a
            return c2

        lax.fori_loop(0, CH, row_body, 0)
        pltpu.sync_copy(out_v, out_hbm.at[pl.ds(base, CH)])
        return carry

    lax.fori_loop(0, NCHUNK, chunk_body, 0)


def _mlp_kernel(interp_ref, ft_ref, w1b_ref, w2_ref, out_ref):
    h = interp_ref[...] + jnp.dot(ft_ref[...], w1b_ref[...],
                                  preferred_element_type=jnp.float32)
    h = jnp.maximum(h, 0.0)
    out = jnp.dot(h, w2_ref[...], preferred_element_type=jnp.float32)
    out_ref[...] = jnp.maximum(out, 0.0)


@jax.jit
def kernel(xyz_target, xyz_source, feats_target, feats_source, W1, W2):
    W1a = W1[:CS]
    W1b = W1[CS:]
    xst = jnp.swapaxes(xyz_source, 1, 2)

    G = pl.pallas_call(
        _g_kernel,
        grid=(B,),
        in_specs=[
            pl.BlockSpec((1, NS, CS), lambda b: (b, 0, 0)),
            pl.BlockSpec((CS, C1), lambda b: (0, 0)),
        ],
        out_specs=pl.BlockSpec((1, NS, C1), lambda b: (b, 0, 0)),
        out_shape=jax.ShapeDtypeStruct((B, NS, C1), jnp.float32),
    )(feats_source, W1a)

    idx3, w3 = pl.pallas_call(
        _nn_kernel,
        grid=(B, NT // TB),
        in_specs=[
            pl.BlockSpec((1, TB, 3), lambda b, t: (b, t, 0)),
            pl.BlockSpec((1, 3, NS), lambda b, t: (b, 0, 0)),
        ],
        out_specs=[
            pl.BlockSpec((1, TB, 3), lambda b, t: (b, t, 0)),
            pl.BlockSpec((1, TB, 48), lambda b, t: (b, t, 0)),
        ],
        out_shape=[
            jax.ShapeDtypeStruct((B, NT, 3), jnp.int32),
            jax.ShapeDtypeStruct((B, NT, 48), jnp.float32),
        ],
    )(xyz_target, xst)

    g_flat = G.reshape(B * NS, C1)
    idx_flat = idx3.reshape(ROWS * 3)
    w_flat = w3.reshape(ROWS * 3, 16)

    sc_gather = pl.kernel(
        _gather_kernel,
        out_type=jax.ShapeDtypeStruct((ROWS, C1), jnp.float32),
        mesh=plsc.VectorSubcoreMesh(core_axis_name="c",
                                    subcore_axis_name="s"),
        scratch_types=[
            pltpu.VMEM((CH * 3,), jnp.int32),
            pltpu.VMEM((CH * 3, 16), jnp.float32),
            pltpu.VMEM((CH * 3, C1), jnp.float32),
            pltpu.VMEM((CH, C1), jnp.float32),
            pltpu.SemaphoreType.DMA,
        ],
    )
    interp = sc_gather(g_flat, idx_flat, w_flat)

    ft_flat = feats_target.reshape(ROWS, CT)
    out = pl.pallas_call(
        _mlp_kernel,
        grid=(ROWS // TBC,),
        in_specs=[
            pl.BlockSpec((TBC, C1), lambda i: (i, 0)),
            pl.BlockSpec((TBC, CT), lambda i: (i, 0)),
            pl.BlockSpec((CT, C1), lambda i: (0, 0)),
            pl.BlockSpec((C1, C2), lambda i: (0, 0)),
        ],
        out_specs=pl.BlockSpec((TBC, C2), lambda i: (i, 0)),
        out_shape=jax.ShapeDtypeStruct((ROWS, C2), jnp.float32),
    )(interp, ft_flat, W1b, W2)
    return out.reshape(B, NT, C2)


# speedup vs baseline: 1.4439x; 1.4439x over previous
import functools

import jax
import jax.numpy as jnp
import numpy as np
from jax.experimental import pallas as pl
from jax.sharding import Mesh, PartitionSpec as P

B, NT, NS = 16, 4096, 1024
CT, CS = 256, 512
C1, C2 = 256, 256
TB = 1024


def _g_kernel(fs_ref, w1a_ref, g_ref):
    g_ref[0] = jnp.dot(fs_ref[0], w1a_ref[...],
                       preferred_element_type=jnp.float32)


def _fp_kernel(xt_ref, xst_ref, ft_ref, g_ref, w1b_ref, w2_ref, out_ref):
    xt = xt_ref[0]
    xst = xst_ref[0]
    diff0 = xt[:, 0:1] - xst[0:1, :]
    d2 = diff0 * diff0
    diff1 = xt[:, 1:2] - xst[1:2, :]
    d2 = d2 + diff1 * diff1
    diff2 = xt[:, 2:3] - xst[2:3, :]
    d2 = d2 + diff2 * diff2

    d2w = d2
    ohs = []
    recips = []
    for k in range(3):
        m = jnp.min(d2w, axis=1, keepdims=True)
        oh = d2w == m
        if k < 2:
            d2w = jnp.where(oh, jnp.float32(jnp.inf), d2w)
        recips.append(jax.lax.rsqrt(jnp.maximum(m, 1e-20)))
        ohs.append(oh)
    r0, r1, r2 = recips
    norm = r0 + r1 + r2
    rn = 1.0 / norm
    ws = (r0 + r1 + r2) * rn + 1e-6
    c = rn / ws
    wsel = jnp.where(ohs[0], r0 * c, 0.0)
    wsel = jnp.where(ohs[1], r1 * c, wsel)
    wsel = jnp.where(ohs[2], r2 * c, wsel)

    interp = jnp.dot(wsel, g_ref[0], preferred_element_type=jnp.float32)
    h = interp + jnp.dot(ft_ref[0], w1b_ref[...],
                         preferred_element_type=jnp.float32)
    h = jnp.maximum(h, 0.0)
    out = jnp.dot(h, w2_ref[...], preferred_element_type=jnp.float32)
    out_ref[0] = jnp.maximum(out, 0.0)


def _impl(xyz_target, xyz_source, feats_target, feats_source, W1, W2):
    Bl = xyz_target.shape[0]
    W1a = W1[:CS]
    W1b = W1[CS:]
    xst = jnp.swapaxes(xyz_source, 1, 2)

    G = pl.pallas_call(
        _g_kernel,
        grid=(Bl,),
        in_specs=[
            pl.BlockSpec((1, NS, CS), lambda b: (b, 0, 0)),
            pl.BlockSpec((CS, C1), lambda b: (0, 0)),
        ],
        out_specs=pl.BlockSpec((1, NS, C1), lambda b: (b, 0, 0)),
        out_shape=jax.ShapeDtypeStruct((Bl, NS, C1), jnp.float32),
    )(feats_source, W1a)

    out = pl.pallas_call(
        _fp_kernel,
        grid=(Bl, NT // TB),
        in_specs=[
            pl.BlockSpec((1, TB, 3), lambda b, t: (b, t, 0)),
            pl.BlockSpec((1, 3, NS), lambda b, t: (b, 0, 0)),
            pl.BlockSpec((1, TB, CT), lambda b, t: (b, t, 0)),
            pl.BlockSpec((1, NS, C1), lambda b, t: (b, 0, 0)),
            pl.BlockSpec((CT, C1), lambda b, t: (0, 0)),
            pl.BlockSpec((C1, C2), lambda b, t: (0, 0)),
        ],
        out_specs=pl.BlockSpec((1, TB, C2), lambda b, t: (b, t, 0)),
        out_shape=jax.ShapeDtypeStruct((Bl, NT, C2), jnp.float32),
    )(xyz_target, xst, feats_target, G, W1b, W2)
    return out


@jax.jit
def kernel(xyz_target, xyz_source, feats_target, feats_source, W1, W2):
    try:
        devs = [d for d in jax.devices() if d.platform == "tpu"]
    except RuntimeError:
        devs = []
    nd = 1
    for cand in (4, 2):
        if len(devs) >= cand and B % cand == 0:
            nd = cand
            break
    if nd == 1:
        return _impl(xyz_target, xyz_source, feats_target, feats_source,
                     W1, W2)
    mesh = Mesh(np.array(devs[:nd]), ("d",))
    f = jax.shard_map(
        _impl, mesh=mesh,
        in_specs=(P("d"), P("d"), P("d"), P("d"), P(), P()),
        out_specs=P("d"), check_vma=False)
    return f(xyz_target, xyz_source, feats_target, feats_source, W1, W2)


# speedup vs baseline: 3.1944x; 2.2124x over previous
import functools

import jax
import jax.numpy as jnp
from jax.experimental import pallas as pl

B, NT, NS = 16, 4096, 1024
CT, CS = 256, 512
C1, C2 = 256, 256
TB = 1024


def _g_kernel(fs_ref, w1a_ref, g_ref):
    g_ref[0] = jnp.dot(fs_ref[0], w1a_ref[...],
                       preferred_element_type=jnp.float32)


def _fp_kernel(xt_ref, xst_ref, ft_ref, g_ref, w1b_ref, w2_ref, out_ref):
    xt = xt_ref[0]
    xst = xst_ref[0]
    diff0 = xt[:, 0:1] - xst[0:1, :]
    d2 = diff0 * diff0
    diff1 = xt[:, 1:2] - xst[1:2, :]
    d2 = d2 + diff1 * diff1
    diff2 = xt[:, 2:3] - xst[2:3, :]
    d2 = d2 + diff2 * diff2

    d2w = d2
    recips = []
    wselu = None
    for k in range(3):
        m = jnp.min(d2w, axis=1, keepdims=True)
        oh = d2w == m
        r = jax.lax.rsqrt(jnp.maximum(m, 1e-20))
        recips.append(r)
        wselu = jnp.where(oh, r, 0.0 if wselu is None else wselu)
        if k < 2:
            d2w = jnp.where(oh, jnp.float32(jnp.inf), d2w)
    r0, r1, r2 = recips
    norm = r0 + r1 + r2
    rn = 1.0 / norm
    ws = (r0 + r1 + r2) * rn + 1e-6
    c = rn / ws

    interp = c * jnp.dot(wselu, g_ref[0], preferred_element_type=jnp.float32)
    h = interp + jnp.dot(ft_ref[0], w1b_ref[...],
                         preferred_element_type=jnp.float32)
    h = jnp.maximum(h, 0.0)
    out = jnp.dot(h, w2_ref[...], preferred_element_type=jnp.float32)
    out_ref[0] = jnp.maximum(out, 0.0)


@jax.jit
def kernel(xyz_target, xyz_source, feats_target, feats_source, W1, W2):
    W1a = W1[:CS]
    W1b = W1[CS:]
    xst = jnp.swapaxes(xyz_source, 1, 2)

    G = pl.pallas_call(
        _g_kernel,
        grid=(B,),
        in_specs=[
            pl.BlockSpec((1, NS, CS), lambda b: (b, 0, 0)),
            pl.BlockSpec((CS, C1), lambda b: (0, 0)),
        ],
        out_specs=pl.BlockSpec((1, NS, C1), lambda b: (b, 0, 0)),
        out_shape=jax.ShapeDtypeStruct((B, NS, C1), jnp.float32),
    )(feats_source, W1a)

    out = pl.pallas_call(
        _fp_kernel,
        grid=(B, NT // TB),
        in_specs=[
            pl.BlockSpec((1, TB, 3), lambda b, t: (b, t, 0)),
            pl.BlockSpec((1, 3, NS), lambda b, t: (b, 0, 0)),
            pl.BlockSpec((1, TB, CT), lambda b, t: (b, t, 0)),
            pl.BlockSpec((1, NS, C1), lambda b, t: (b, 0, 0)),
            pl.BlockSpec((CT, C1), lambda b, t: (0, 0)),
            pl.BlockSpec((C1, C2), lambda b, t: (0, 0)),
        ],
        out_specs=pl.BlockSpec((1, TB, C2), lambda b, t: (b, t, 0)),
        out_shape=jax.ShapeDtypeStruct((B, NT, C2), jnp.float32),
    )(xyz_target, xst, feats_target, G, W1b, W2)
    return out


# speedup vs baseline: 3.3586x; 1.0514x over previous
import functools

import jax
import jax.numpy as jnp
from jax.experimental import pallas as pl

B, NT, NS = 16, 4096, 1024
CT, CS = 256, 512
C1, C2 = 256, 256
TB = 2048


def _g_kernel(fs_ref, w1a_ref, g_ref):
    g_ref[0] = jnp.dot(fs_ref[0], w1a_ref[...],
                       preferred_element_type=jnp.float32)


def _fp_kernel(xt_ref, xst_ref, ft_ref, g_ref, w1b_ref, w2_ref, out_ref):
    xt = xt_ref[0]
    xst = xst_ref[0]
    diff0 = xt[:, 0:1] - xst[0:1, :]
    d2 = diff0 * diff0
    diff1 = xt[:, 1:2] - xst[1:2, :]
    d2 = d2 + diff1 * diff1
    diff2 = xt[:, 2:3] - xst[2:3, :]
    d2 = d2 + diff2 * diff2

    d2w = d2
    recips = []
    wselu = None
    for k in range(3):
        m = jnp.min(d2w, axis=1, keepdims=True)
        oh = d2w == m
        r = jax.lax.rsqrt(jnp.maximum(m, 1e-20))
        recips.append(r)
        wselu = jnp.where(oh, r, 0.0 if wselu is None else wselu)
        if k < 2:
            d2w = jnp.where(oh, jnp.float32(jnp.inf), d2w)
    r0, r1, r2 = recips
    norm = r0 + r1 + r2
    rn = 1.0 / norm
    ws = (r0 + r1 + r2) * rn + 1e-6
    c = rn / ws

    interp = c * jnp.dot(wselu, g_ref[0], preferred_element_type=jnp.float32)
    h = interp + jnp.dot(ft_ref[0], w1b_ref[...],
                         preferred_element_type=jnp.float32)
    h = jnp.maximum(h, 0.0)
    out = jnp.dot(h, w2_ref[...], preferred_element_type=jnp.float32)
    out_ref[0] = jnp.maximum(out, 0.0)


@jax.jit
def kernel(xyz_target, xyz_source, feats_target, feats_source, W1, W2):
    W1a = W1[:CS]
    W1b = W1[CS:]
    xst = jnp.swapaxes(xyz_source, 1, 2)

    G = pl.pallas_call(
        _g_kernel,
        grid=(B,),
        in_specs=[
            pl.BlockSpec((1, NS, CS), lambda b: (b, 0, 0)),
            pl.BlockSpec((CS, C1), lambda b: (0, 0)),
        ],
        out_specs=pl.BlockSpec((1, NS, C1), lambda b: (b, 0, 0)),
        out_shape=jax.ShapeDtypeStruct((B, NS, C1), jnp.float32),
    )(feats_source, W1a)

    out = pl.pallas_call(
        _fp_kernel,
        grid=(B, NT // TB),
        in_specs=[
            pl.BlockSpec((1, TB, 3), lambda b, t: (b, t, 0)),
            pl.BlockSpec((1, 3, NS), lambda b, t: (b, 0, 0)),
            pl.BlockSpec((1, TB, CT), lambda b, t: (b, t, 0)),
            pl.BlockSpec((1, NS, C1), lambda b, t: (b, 0, 0)),
            pl.BlockSpec((CT, C1), lambda b, t: (0, 0)),
            pl.BlockSpec((C1, C2), lambda b, t: (0, 0)),
        ],
        out_specs=pl.BlockSpec((1, TB, C2), lambda b, t: (b, t, 0)),
        out_shape=jax.ShapeDtypeStruct((B, NT, C2), jnp.float32),
    )(xyz_target, xst, feats_target, G, W1b, W2)
    return out


# speedup vs baseline: 3.4886x; 1.0387x over previous
import functools

import jax
import jax.numpy as jnp
from jax.experimental import pallas as pl
from jax.experimental.pallas import tpu as pltpu

B, NT, NS = 16, 4096, 1024
CT, CS = 256, 512
C1, C2 = 256, 256
TB = 2048


def _fp_kernel(xt_ref, xst_ref, ft_ref, fs_ref, w1a_ref, w1b_ref, w2_ref,
               out_ref, g_scr):
    @pl.when(pl.program_id(1) == 0)
    def _():
        g_scr[...] = jnp.dot(fs_ref[0], w1a_ref[...],
                             preferred_element_type=jnp.float32)
    xt = xt_ref[0]
    xst = xst_ref[0]
    diff0 = xt[:, 0:1] - xst[0:1, :]
    d2 = diff0 * diff0
    diff1 = xt[:, 1:2] - xst[1:2, :]
    d2 = d2 + diff1 * diff1
    diff2 = xt[:, 2:3] - xst[2:3, :]
    d2 = d2 + diff2 * diff2

    d2w = d2
    recips = []
    wselu = None
    for k in range(3):
        m = jnp.min(d2w, axis=1, keepdims=True)
        oh = d2w == m
        r = jax.lax.rsqrt(jnp.maximum(m, 1e-20))
        recips.append(r)
        wselu = jnp.where(oh, r, 0.0 if wselu is None else wselu)
        if k < 2:
            d2w = jnp.where(oh, jnp.float32(jnp.inf), d2w)
    r0, r1, r2 = recips
    norm = r0 + r1 + r2
    rn = 1.0 / norm
    ws = (r0 + r1 + r2) * rn + 1e-6
    c = rn / ws

    interp = c * jnp.dot(wselu, g_scr[...],
                         preferred_element_type=jnp.float32)
    h = interp + jnp.dot(ft_ref[0], w1b_ref[...],
                         preferred_element_type=jnp.float32)
    h = jnp.maximum(h, 0.0)
    out = jnp.dot(h, w2_ref[...], preferred_element_type=jnp.float32)
    out_ref[0] = jnp.maximum(out, 0.0)


@jax.jit
def kernel(xyz_target, xyz_source, feats_target, feats_source, W1, W2):
    W1a = W1[:CS]
    W1b = W1[CS:]
    xst = jnp.swapaxes(xyz_source, 1, 2)

    out = pl.pallas_call(
        _fp_kernel,
        grid=(B, NT // TB),
        in_specs=[
            pl.BlockSpec((1, TB, 3), lambda b, t: (b, t, 0)),
            pl.BlockSpec((1, 3, NS), lambda b, t: (b, 0, 0)),
            pl.BlockSpec((1, TB, CT), lambda b, t: (b, t, 0)),
            pl.BlockSpec((1, NS, CS), lambda b, t: (b, 0, 0)),
            pl.BlockSpec((CS, C1), lambda b, t: (0, 0)),
            pl.BlockSpec((CT, C1), lambda b, t: (0, 0)),
            pl.BlockSpec((C1, C2), lambda b, t: (0, 0)),
        ],
        out_specs=pl.BlockSpec((1, TB, C2), lambda b, t: (b, t, 0)),
        out_shape=jax.ShapeDtypeStruct((B, NT, C2), jnp.float32),
        scratch_shapes=[pltpu.VMEM((NS, C1), jnp.float32)],
    )(xyz_target, xst, feats_target, feats_source, W1a, W1b, W2)
    return out
